# baseline (speedup 1.0000x reference)
import math

import numpy as np
import jax
import jax.numpy as jnp
from jax.experimental import pallas as pl
from jax.experimental.pallas import tpu as pltpu

_D_MODEL = 768
_MAX_LEN = 1600


def _make_pe_np():
    pe = np.zeros((_MAX_LEN, _D_MODEL), dtype=np.float32)
    position = np.arange(0, _MAX_LEN, dtype=np.float32)[:, None]
    div_term = np.exp(
        np.arange(0, _D_MODEL, 2, dtype=np.float32) * (-math.log(10000.0) / _D_MODEL)
    )
    pe[:, 0::2] = np.sin(position * div_term)
    pe[:, 1::2] = np.cos(position * div_term)
    return pe


def _top3_kernel(first_ref, out_ref):
    f = first_ref[:, 0, :]
    s = f.shape[0]
    soft = jnp.sum(jnp.abs(f[:, None, :] - f[None, :, :]), axis=-1)
    col = jax.lax.broadcasted_iota(jnp.int32, (s, s), 1)
    big = jnp.int32(2**30)
    for k in range(3):
        minv = jnp.min(soft, axis=1, keepdims=True)
        idx = jnp.min(jnp.where(soft == minv, col, big), axis=1)
        out_ref[:, k : k + 1] = idx[:, None]
        soft = jnp.where(col == idx[:, None], jnp.inf, soft)


def _attn_kernel(idx_ref, in_ref, pe_ref, out_ref, rinv_ref, xb_ref):
    i = pl.program_id(0)
    w = in_ref.shape[1]
    n_per = out_ref.shape[0] // w
    h = in_ref.shape[2]
    scale = 1.0 / math.sqrt(h)
    for j in range(n_per):
        for s in range(3):
            xs = in_ref[idx_ref[n_per * i + j, s]] + pe_ref[s]
            xb_ref[j, s * w : (s + 1) * w] = xs.astype(jnp.bfloat16)
    for j in range(n_per):
        xb = xb_ref[j]
        q = xb[:w]
        scores = jax.lax.dot_general(
            q, xb, (((1,), (1,)), ((), ())), preferred_element_type=jnp.float32
        )
        scores = scores * scale
        m = jnp.max(scores, axis=1, keepdims=True)
        e = jnp.exp(scores - m)
        rinv = 1.0 / jnp.sum(e, axis=1, keepdims=True)
        eb = e.astype(jnp.bfloat16)
        out = jax.lax.dot_general(
            eb, xb, (((1,), (0,)), ((), ())), preferred_element_type=jnp.float32
        )
        out_ref[j * w : (j + 1) * w] = out
        rinv_ref[j * w : (j + 1) * w] = rinv


def kernel(inputs):
    sentence, word, hidden = inputs.shape

    top3 = pl.pallas_call(
        _top3_kernel,
        grid=(1,),
        in_specs=[pl.BlockSpec((sentence, 8, hidden), lambda i: (0, 0, 0))],
        out_specs=pl.BlockSpec((sentence, 128), lambda i: (0, 0)),
        out_shape=jax.ShapeDtypeStruct((sentence, 128), jnp.int32),
    )(inputs)

    pe3 = jnp.asarray(_make_pe_np()[: 3 * word].reshape(3, word, hidden))

    n_per = 8
    grid_spec = pltpu.PrefetchScalarGridSpec(
        num_scalar_prefetch=1,
        grid=(sentence // n_per,),
        in_specs=[
            pl.BlockSpec((sentence, word, hidden), lambda i, idx: (0, 0, 0)),
            pl.BlockSpec((3, word, hidden), lambda i, idx: (0, 0, 0)),
        ],
        out_specs=[
            pl.BlockSpec((n_per * word, hidden), lambda i, idx: (i, 0)),
            pl.BlockSpec((n_per * word, 1), lambda i, idx: (i, 0)),
        ],
        scratch_shapes=[pltpu.VMEM((n_per, 3 * word, hidden), jnp.bfloat16)],
    )
    flat, rinv = pl.pallas_call(
        _attn_kernel,
        grid_spec=grid_spec,
        out_shape=[
            jax.ShapeDtypeStruct((sentence * word, hidden), jnp.float32),
            jax.ShapeDtypeStruct((sentence * word, 1), jnp.float32),
        ],
    )(top3, inputs, pe3)
    unnorm = flat.reshape(sentence, word, hidden)[:, : word - 1, :]
    scalev = rinv.reshape(sentence, word, 1)[:, : word - 1, :]
    return unnorm * scalev


# speedup vs baseline: 1.1541x; 1.1541x over previous
import math

import numpy as np
import jax
import jax.numpy as jnp
from jax.experimental import pallas as pl
from jax.experimental.pallas import tpu as pltpu

_D_MODEL = 768
_MAX_LEN = 1600


def _make_pe_np():
    pe = np.zeros((_MAX_LEN, _D_MODEL), dtype=np.float32)
    position = np.arange(0, _MAX_LEN, dtype=np.float32)[:, None]
    div_term = np.exp(
        np.arange(0, _D_MODEL, 2, dtype=np.float32) * (-math.log(10000.0) / _D_MODEL)
    )
    pe[:, 0::2] = np.sin(position * div_term)
    pe[:, 1::2] = np.cos(position * div_term)
    return pe


def _top3_kernel(first_ref, out_ref):
    f = first_ref[:, 0, :]
    s = f.shape[0]
    soft = jnp.sum(jnp.abs(f[:, None, :] - f[None, :, :]), axis=-1)
    col = jax.lax.broadcasted_iota(jnp.int32, (s, s), 1)
    big = jnp.int32(2**30)
    for k in range(3):
        minv = jnp.min(soft, axis=1, keepdims=True)
        idx = jnp.min(jnp.where(soft == minv, col, big), axis=1)
        out_ref[:, k : k + 1] = idx[:, None]
        soft = jnp.where(col == idx[:, None], jnp.inf, soft)


def _attn_kernel(idx_ref, in_ref, pe_ref, out_ref, xb_ref):
    i = pl.program_id(0)
    w = in_ref.shape[1]
    n_per = out_ref.shape[0]
    h = in_ref.shape[2]
    scale = 1.0 / math.sqrt(h)
    for j in range(n_per):
        for s in range(3):
            xs = in_ref[idx_ref[n_per * i + j, s]] + pe_ref[s]
            xb_ref[j, s * w : (s + 1) * w] = xs.astype(jnp.bfloat16)
    for j in range(n_per):
        xb = xb_ref[j]
        q = xb[:w]
        scores = jax.lax.dot_general(
            q, xb, (((1,), (1,)), ((), ())), preferred_element_type=jnp.float32
        )
        scores = scores * scale
        m = jnp.max(scores, axis=1, keepdims=True)
        e = jnp.exp(scores - m)
        rinv = 1.0 / jnp.sum(e, axis=1, keepdims=True)
        eb = e.astype(jnp.bfloat16)
        out = jax.lax.dot_general(
            eb, xb, (((1,), (0,)), ((), ())), preferred_element_type=jnp.float32
        )
        out_ref[j] = (out * rinv)[: out_ref.shape[1]]


def kernel(inputs):
    sentence, word, hidden = inputs.shape

    top3 = pl.pallas_call(
        _top3_kernel,
        grid=(1,),
        in_specs=[pl.BlockSpec((sentence, 8, hidden), lambda i: (0, 0, 0))],
        out_specs=pl.BlockSpec((sentence, 128), lambda i: (0, 0)),
        out_shape=jax.ShapeDtypeStruct((sentence, 128), jnp.int32),
    )(inputs)

    pe3 = jnp.asarray(_make_pe_np()[: 3 * word].reshape(3, word, hidden))

    n_per = 8
    grid_spec = pltpu.PrefetchScalarGridSpec(
        num_scalar_prefetch=1,
        grid=(sentence // n_per,),
        in_specs=[
            pl.BlockSpec((sentence, word, hidden), lambda i, idx: (0, 0, 0)),
            pl.BlockSpec((3, word, hidden), lambda i, idx: (0, 0, 0)),
        ],
        out_specs=pl.BlockSpec((n_per, word - 1, hidden), lambda i, idx: (i, 0, 0)),
        scratch_shapes=[pltpu.VMEM((n_per, 3 * word, hidden), jnp.bfloat16)],
    )
    return pl.pallas_call(
        _attn_kernel,
        grid_spec=grid_spec,
        out_shape=jax.ShapeDtypeStruct((sentence, word - 1, hidden), jnp.float32),
    )(top3, inputs, pe3)
